# Initial kernel scaffold; baseline (speedup 1.0000x reference)
import jax
import jax.numpy as jnp
from jax.experimental import pallas as pl


def kernel(edge_index, subj, rel, edge_type, edge_norm, init_embed, init_rel, w_in_0, w_out_0, w_loop_0, w_rel_0, loop_rel_0, bias_0, gamma_0, beta_0, w_in_1, w_out_1, w_loop_1, w_rel_1, loop_rel_1, bias_1, gamma_1, beta_1):
    raise NotImplementedError("write your pallas kernel here")


# speedup vs baseline: 3.5493x; 3.5493x over previous
import functools

import jax
import jax.numpy as jnp
from jax import lax
from jax.experimental import pallas as pl
from jax.experimental.pallas import tpu as pltpu
from jax.experimental.pallas import tpu_sc as plsc

NUM_ENT = 10000
NUM_REL2 = 400
DIM = 128
N_EDGES = 320000
HALF = N_EDGES // 2
BATCH = 1024

NC = 2
NS = 16
LN = 16

EDGES_PER_TILE = HALF // NS
CHUNK = 128
NFULL = EDGES_PER_TILE // CHUNK
REM = EDGES_PER_TILE - NFULL * CHUNK
ACC_ROWS = 10240
ROWS_PER_TILE = ACC_ROWS // NS
ZROWS = 128

_SC_MESH = plsc.VectorSubcoreMesh(core_axis_name="c", subcore_axis_name="s")


def _edge_agg_body(src_hbm, dst_hbm, et_hbm, norm_hbm, x_hbm, r_hbm, acc_hbm,
                   src_v, dst_v, et_v, norm_v, xrows_v, rrows_v,
                   src_t, dst_t, et_t, norm_t, xrows_t, rrows_t, acc_sh,
                   sem_x, sem_r):
    c = lax.axis_index("c")
    s = lax.axis_index("s")

    zero = jnp.zeros((LN,), jnp.float32)

    def _zrow(i, _):
        for j in range(DIM // LN):
            xrows_v[i, pl.ds(j * LN, LN)] = zero
        return 0

    lax.fori_loop(0, ZROWS, _zrow, 0)
    for m in range(ROWS_PER_TILE // ZROWS):
        pltpu.sync_copy(xrows_v, acc_sh.at[pl.ds(s * ROWS_PER_TILE + m * ZROWS, ZROWS)])

    plsc.subcore_barrier()

    ebase = c * HALF + s * EDGES_PER_TILE

    def _chunk(base, n, srcb, dstb, etb, normb, xrows, rrows):
        pltpu.sync_copy(src_hbm.at[pl.ds(base, n)], srcb)
        pltpu.sync_copy(dst_hbm.at[pl.ds(base, n)], dstb)
        pltpu.sync_copy(et_hbm.at[pl.ds(base, n)], etb)
        pltpu.sync_copy(norm_hbm.at[pl.ds(base, n)], normb)
        cp_x = pltpu.async_copy(x_hbm.at[srcb], xrows, sem_x)
        cp_r = pltpu.async_copy(r_hbm.at[etb], rrows, sem_r)
        cp_x.wait()
        cp_r.wait()

        def _group(g, _):
            nr16 = normb[pl.ds(g * LN, LN)]
            for li in range(LN):
                nrm = nr16[li]
                i = g * LN + li
                for j in range(DIM // LN):
                    sl = pl.ds(j * LN, LN)
                    xrows[i, sl] = xrows[i, sl] * rrows[i, sl] * nrm
            return 0

        lax.fori_loop(0, n // LN, _group, 0)
        pltpu.sync_copy(xrows, acc_sh.at[dstb], add=True)

    def _full_chunk(k, _):
        _chunk(ebase + k * CHUNK, CHUNK,
               src_v, dst_v, et_v, norm_v, xrows_v, rrows_v)
        return 0

    lax.fori_loop(0, NFULL, _full_chunk, 0)
    if REM:
        _chunk(ebase + NFULL * CHUNK, REM,
               src_t, dst_t, et_t, norm_t, xrows_t, rrows_t)

    plsc.subcore_barrier()
    pltpu.sync_copy(acc_sh.at[pl.ds(s * ROWS_PER_TILE, ROWS_PER_TILE)],
                    acc_hbm.at[c].at[pl.ds(s * ROWS_PER_TILE, ROWS_PER_TILE)])


_edge_agg = pl.kernel(
    _edge_agg_body,
    out_type=jax.ShapeDtypeStruct((NC, ACC_ROWS, DIM), jnp.float32),
    mesh=_SC_MESH,
    scratch_types=[
        pltpu.VMEM((CHUNK,), jnp.int32),
        pltpu.VMEM((CHUNK,), jnp.int32),
        pltpu.VMEM((CHUNK,), jnp.int32),
        pltpu.VMEM((CHUNK,), jnp.float32),
        pltpu.VMEM((CHUNK, DIM), jnp.float32),
        pltpu.VMEM((CHUNK, DIM), jnp.float32),
        pltpu.VMEM((REM,), jnp.int32),
        pltpu.VMEM((REM,), jnp.int32),
        pltpu.VMEM((REM,), jnp.int32),
        pltpu.VMEM((REM,), jnp.float32),
        pltpu.VMEM((REM, DIM), jnp.float32),
        pltpu.VMEM((REM, DIM), jnp.float32),
        pltpu.VMEM_SHARED((ACC_ROWS, DIM), jnp.float32),
        pltpu.SemaphoreType.DMA,
        pltpu.SemaphoreType.DMA,
    ],
)


def _dense_body(acc_ref, x_ref, win_ref, wout_ref, wloop_ref, lrel_ref,
                bias_ref, gamma_ref, beta_ref, r_ref, wrel_ref,
                xout_ref, rout_ref):
    a_in = acc_ref[0]
    a_out = acc_ref[1]
    x = x_ref[...]
    pre = (jnp.dot(a_in, win_ref[...], preferred_element_type=jnp.float32)
           + jnp.dot(a_out, wout_ref[...], preferred_element_type=jnp.float32)
           + jnp.dot(x * lrel_ref[...], wloop_ref[...],
                     preferred_element_type=jnp.float32)) * (1.0 / 3.0)
    pre = pre + bias_ref[...]
    mean = jnp.mean(pre, axis=0, keepdims=True)
    var = jnp.mean((pre - mean) ** 2, axis=0, keepdims=True)
    xout_ref[...] = ((pre - mean) * lax.rsqrt(var + 1e-5) * gamma_ref[...]
                     + beta_ref[...])
    rout_ref[...] = jnp.dot(r_ref[...], wrel_ref[...],
                            preferred_element_type=jnp.float32)


_dense = pl.pallas_call(
    _dense_body,
    out_shape=(jax.ShapeDtypeStruct((NUM_ENT, DIM), jnp.float32),
               jax.ShapeDtypeStruct((NUM_REL2, DIM), jnp.float32)),
)

B_PER_W = BATCH // (NC * NS)


def _lookup_body(x_hbm, r_hbm, subj_hbm, rel_hbm, sub_out, rel_out,
                 idx_v, rows_v, sem):
    c = lax.axis_index("c")
    s = lax.axis_index("s")
    base = (s * NC + c) * B_PER_W
    pltpu.sync_copy(subj_hbm.at[pl.ds(base, B_PER_W)], idx_v)
    pltpu.async_copy(x_hbm.at[idx_v], rows_v, sem).wait()
    pltpu.sync_copy(rows_v, sub_out.at[pl.ds(base, B_PER_W)])
    pltpu.sync_copy(rel_hbm.at[pl.ds(base, B_PER_W)], idx_v)
    pltpu.async_copy(r_hbm.at[idx_v], rows_v, sem).wait()
    pltpu.sync_copy(rows_v, rel_out.at[pl.ds(base, B_PER_W)])


_lookup = pl.kernel(
    _lookup_body,
    out_type=(jax.ShapeDtypeStruct((BATCH, DIM), jnp.float32),
              jax.ShapeDtypeStruct((BATCH, DIM), jnp.float32)),
    mesh=_SC_MESH,
    scratch_types=[
        pltpu.VMEM((B_PER_W,), jnp.int32),
        pltpu.VMEM((B_PER_W, DIM), jnp.float32),
        pltpu.SemaphoreType.DMA,
    ],
)


def kernel(edge_index, subj, rel, edge_type, edge_norm, init_embed, init_rel,
           w_in_0, w_out_0, w_loop_0, w_rel_0, loop_rel_0, bias_0, gamma_0, beta_0,
           w_in_1, w_out_1, w_loop_1, w_rel_1, loop_rel_1, bias_1, gamma_1, beta_1):
    src, dst = edge_index[0], edge_index[1]
    x, r = init_embed, init_rel
    for (w_in, w_out, w_loop, w_rel, loop_rel, bias, gamma, beta) in (
            (w_in_0, w_out_0, w_loop_0, w_rel_0, loop_rel_0, bias_0, gamma_0, beta_0),
            (w_in_1, w_out_1, w_loop_1, w_rel_1, loop_rel_1, bias_1, gamma_1, beta_1)):
        acc = _edge_agg(src, dst, edge_type, edge_norm, x, r)[:, :NUM_ENT]
        x, r = _dense(acc, x, w_in, w_out, w_loop, loop_rel,
                      bias.reshape(1, DIM), gamma.reshape(1, DIM),
                      beta.reshape(1, DIM), r, w_rel)
    sub_emb, rel_emb = _lookup(x, r, subj, rel)
    return (sub_emb, rel_emb, x)


# speedup vs baseline: 5.3046x; 1.4946x over previous
import jax
import jax.numpy as jnp
from jax import lax
from jax.experimental import pallas as pl
from jax.experimental.pallas import tpu as pltpu
from jax.experimental.pallas import tpu_sc as plsc

NUM_ENT = 10000
NUM_REL2 = 400
DIM = 128
N_EDGES = 320000
HALF = N_EDGES // 2
BATCH = 1024

NC = 2
NS = 16
LN = 16

EDGES_PER_TILE = HALF // NS
CHUNK = 48
NFULL = EDGES_PER_TILE // CHUNK
REM = EDGES_PER_TILE - NFULL * CHUNK
NGRP = CHUNK // LN
ACC_ROWS = 10112
ROWS_PER_TILE = ACC_ROWS // NS

_SC_MESH = plsc.VectorSubcoreMesh(core_axis_name="c", subcore_axis_name="s")


def _mul_chunk(xrows, rrows, normb, n):
    def _group(g, _):
        nr16 = normb[pl.ds(g * LN, LN)]
        for li in range(LN):
            nrm = nr16[li]
            i = g * LN + li
            for j in range(DIM // LN):
                sl = pl.ds(j * LN, LN)
                xrows[i, sl] = xrows[i, sl] * rrows[i, sl] * nrm
        return 0

    lax.fori_loop(0, n // LN, _group, 0)


def _edge_agg_body(src_hbm, et_hbm, norm_hbm, dst_hbm, x_hbm, r_hbm, acc_hbm,
                   src0, src1, src2, et0, et1, et2, nb0, nb1, nb2,
                   db0, db1, db2, xr0, xr1, xr2, rr0, rr1, rr2,
                   src_t, et_t, nb_t, dst_t,
                   acc_sh,
                   si0, si1, si2, sd0, sd1, sd2,
                   sx0, sx1, sx2, sr0, sr1, sr2, ss0, ss1, ss2):
    c = lax.axis_index("c")
    s = lax.axis_index("s")
    srcb = (src0, src1, src2)
    etb = (et0, et1, et2)
    nbb = (nb0, nb1, nb2)
    dbb = (db0, db1, db2)
    xrb = (xr0, xr1, xr2)
    rrb = (rr0, rr1, rr2)
    sib = (si0, si1, si2)
    sdb = (sd0, sd1, sd2)
    sxb = (sx0, sx1, sx2)
    srb = (sr0, sr1, sr2)
    ssb = (ss0, ss1, ss2)

    zero = jnp.zeros((LN,), jnp.float32)

    def _zrow(i, _):
        for j in range(DIM // LN):
            xr0[i, pl.ds(j * LN, LN)] = zero
        return 0

    lax.fori_loop(0, CHUNK, _zrow, 0)
    rbase = s * ROWS_PER_TILE
    for m in range(ROWS_PER_TILE // CHUNK):
        pltpu.sync_copy(xr0, acc_sh.at[pl.ds(rbase + m * CHUNK, CHUNK)])
    pltpu.sync_copy(xr0.at[pl.ds(0, ROWS_PER_TILE % CHUNK)],
                    acc_sh.at[pl.ds(rbase + (ROWS_PER_TILE // CHUNK) * CHUNK,
                                    ROWS_PER_TILE % CHUNK)])
    plsc.subcore_barrier()

    ebase = c * HALF + s * EDGES_PER_TILE

    def _issue_idx(k, sl):
        base = ebase + k * CHUNK
        pltpu.async_copy(src_hbm.at[pl.ds(base, CHUNK)], srcb[sl], sib[sl])
        pltpu.async_copy(et_hbm.at[pl.ds(base, CHUNK)], etb[sl], sib[sl])
        pltpu.async_copy(norm_hbm.at[pl.ds(base, CHUNK)], nbb[sl], sib[sl])

    def _wait_idx(sl):
        pltpu.make_async_copy(src_hbm.at[pl.ds(0, CHUNK)], srcb[sl], sib[sl]).wait()
        pltpu.make_async_copy(et_hbm.at[pl.ds(0, CHUNK)], etb[sl], sib[sl]).wait()
        pltpu.make_async_copy(norm_hbm.at[pl.ds(0, CHUNK)], nbb[sl], sib[sl]).wait()

    def _issue_dst(k, sl):
        pltpu.async_copy(dst_hbm.at[pl.ds(ebase + k * CHUNK, CHUNK)],
                         dbb[sl], sdb[sl])

    def _wait_dst(sl):
        pltpu.make_async_copy(dst_hbm.at[pl.ds(0, CHUNK)], dbb[sl], sdb[sl]).wait()

    def _issue_gathers(sl):
        pltpu.async_copy(x_hbm.at[srcb[sl]], xrb[sl], sxb[sl])
        pltpu.async_copy(r_hbm.at[etb[sl]], rrb[sl], srb[sl])

    def _wait_gathers(sl):
        pltpu.make_async_copy(x_hbm.at[srcb[sl]], xrb[sl], sxb[sl]).wait()
        pltpu.make_async_copy(r_hbm.at[etb[sl]], rrb[sl], srb[sl]).wait()

    def _issue_scatter(sl):
        pltpu.async_copy(xrb[sl], acc_sh.at[dbb[sl]], ssb[sl], add=True)

    def _wait_scatter(sl):
        pltpu.make_async_copy(xrb[sl], acc_sh.at[dbb[sl]], ssb[sl]).wait()

    _issue_idx(0, 0)
    _issue_idx(1, 1)
    _issue_dst(0, 0)
    _wait_idx(0)
    _issue_gathers(0)

    def _triple(it, _):
        for jj in range(3):
            s0 = jj
            s1 = (jj + 1) % 3
            s2 = (jj + 2) % 3
            kabs = 3 * it + jj

            @pl.when(kabs >= 2)
            def _():
                _wait_scatter(s1)

            @pl.when(kabs + 1 < NFULL)
            def _():
                _issue_dst(kabs + 1, s1)
                _wait_idx(s1)
                _issue_gathers(s1)

            @pl.when(kabs + 2 < NFULL)
            def _():
                _issue_idx(kabs + 2, s2)

            _wait_gathers(s0)
            _mul_chunk(xrb[s0], rrb[s0], nbb[s0], CHUNK)
            _wait_dst(s0)
            _issue_scatter(s0)
        return 0

    NCOV = (NFULL // 3) * 3
    lax.fori_loop(0, NFULL // 3, _triple, 0)

    _wait_scatter((NCOV - 2) % 3)
    _wait_scatter((NCOV - 1) % 3)

    for k in range(NCOV, NFULL):
        sl = k % 3
        _wait_gathers(sl)
        _mul_chunk(xrb[sl], rrb[sl], nbb[sl], CHUNK)
        _wait_dst(sl)
        _issue_scatter(sl)
        _wait_scatter(sl)

    tbase = ebase + NFULL * CHUNK
    pltpu.sync_copy(src_hbm.at[pl.ds(tbase, REM)], src_t)
    pltpu.sync_copy(et_hbm.at[pl.ds(tbase, REM)], et_t)
    pltpu.sync_copy(norm_hbm.at[pl.ds(tbase, REM)], nb_t)
    pltpu.sync_copy(dst_hbm.at[pl.ds(tbase, REM)], dst_t)
    pltpu.async_copy(x_hbm.at[src_t], xr0.at[pl.ds(0, REM)], sx0).wait()
    pltpu.async_copy(r_hbm.at[et_t], rr0.at[pl.ds(0, REM)], sr0).wait()
    _mul_chunk(xr0, rr0, nb_t, REM)
    pltpu.sync_copy(xr0.at[pl.ds(0, REM)], acc_sh.at[dst_t], add=True)

    plsc.subcore_barrier()
    pltpu.sync_copy(acc_sh.at[pl.ds(rbase, ROWS_PER_TILE)],
                    acc_hbm.at[c].at[pl.ds(rbase, ROWS_PER_TILE)])


_edge_agg = pl.kernel(
    _edge_agg_body,
    out_type=jax.ShapeDtypeStruct((NC, ACC_ROWS, DIM), jnp.float32),
    mesh=_SC_MESH,
    scratch_types=(
        [pltpu.VMEM((CHUNK,), jnp.int32) for _ in range(3)]
        + [pltpu.VMEM((CHUNK,), jnp.int32) for _ in range(3)]
        + [pltpu.VMEM((CHUNK,), jnp.float32) for _ in range(3)]
        + [pltpu.VMEM((CHUNK,), jnp.int32) for _ in range(3)]
        + [pltpu.VMEM((CHUNK, DIM), jnp.float32) for _ in range(3)]
        + [pltpu.VMEM((CHUNK, DIM), jnp.float32) for _ in range(3)]
        + [pltpu.VMEM((REM,), jnp.int32) for _ in range(2)]
        + [pltpu.VMEM((REM,), jnp.float32)]
        + [pltpu.VMEM((REM,), jnp.int32)]
        + [pltpu.VMEM_SHARED((ACC_ROWS, DIM), jnp.float32)]
        + [pltpu.SemaphoreType.DMA for _ in range(15)]
    ),
)


def _dense_body(acc_ref, x_ref, win_ref, wout_ref, wloop_ref, lrel_ref,
                bias_ref, gamma_ref, beta_ref, r_ref, wrel_ref,
                xout_ref, rout_ref):
    a_in = acc_ref[0]
    a_out = acc_ref[1]
    x = x_ref[...]
    pre = (jnp.dot(a_in, win_ref[...], preferred_element_type=jnp.float32)
           + jnp.dot(a_out, wout_ref[...], preferred_element_type=jnp.float32)
           + jnp.dot(x * lrel_ref[...], wloop_ref[...],
                     preferred_element_type=jnp.float32)) * (1.0 / 3.0)
    pre = pre + bias_ref[...]
    mean = jnp.mean(pre, axis=0, keepdims=True)
    var = jnp.mean((pre - mean) ** 2, axis=0, keepdims=True)
    xout_ref[...] = ((pre - mean) * lax.rsqrt(var + 1e-5) * gamma_ref[...]
                     + beta_ref[...])
    rout_ref[...] = jnp.dot(r_ref[...], wrel_ref[...],
                            preferred_element_type=jnp.float32)


_dense = pl.pallas_call(
    _dense_body,
    out_shape=(jax.ShapeDtypeStruct((NUM_ENT, DIM), jnp.float32),
               jax.ShapeDtypeStruct((NUM_REL2, DIM), jnp.float32)),
)

B_PER_W = BATCH // (NC * NS)


def _lookup_body(x_hbm, r_hbm, subj_hbm, rel_hbm, sub_out, rel_out,
                 idx_v, rows_v, sem):
    c = lax.axis_index("c")
    s = lax.axis_index("s")
    base = (s * NC + c) * B_PER_W
    pltpu.sync_copy(subj_hbm.at[pl.ds(base, B_PER_W)], idx_v)
    pltpu.async_copy(x_hbm.at[idx_v], rows_v, sem).wait()
    pltpu.sync_copy(rows_v, sub_out.at[pl.ds(base, B_PER_W)])
    pltpu.sync_copy(rel_hbm.at[pl.ds(base, B_PER_W)], idx_v)
    pltpu.async_copy(r_hbm.at[idx_v], rows_v, sem).wait()
    pltpu.sync_copy(rows_v, rel_out.at[pl.ds(base, B_PER_W)])


_lookup = pl.kernel(
    _lookup_body,
    out_type=(jax.ShapeDtypeStruct((BATCH, DIM), jnp.float32),
              jax.ShapeDtypeStruct((BATCH, DIM), jnp.float32)),
    mesh=_SC_MESH,
    scratch_types=[
        pltpu.VMEM((B_PER_W,), jnp.int32),
        pltpu.VMEM((B_PER_W, DIM), jnp.float32),
        pltpu.SemaphoreType.DMA,
    ],
)


def kernel(edge_index, subj, rel, edge_type, edge_norm, init_embed, init_rel,
           w_in_0, w_out_0, w_loop_0, w_rel_0, loop_rel_0, bias_0, gamma_0, beta_0,
           w_in_1, w_out_1, w_loop_1, w_rel_1, loop_rel_1, bias_1, gamma_1, beta_1):
    src, dst = edge_index[0], edge_index[1]
    x, r = init_embed, init_rel
    for (w_in, w_out, w_loop, w_rel, loop_rel, bias, gamma, beta) in (
            (w_in_0, w_out_0, w_loop_0, w_rel_0, loop_rel_0, bias_0, gamma_0, beta_0),
            (w_in_1, w_out_1, w_loop_1, w_rel_1, loop_rel_1, bias_1, gamma_1, beta_1)):
        acc = _edge_agg(src, edge_type, edge_norm, dst, x, r)[:, :NUM_ENT]
        x, r = _dense(acc, x, w_in, w_out, w_loop, loop_rel,
                      bias.reshape(1, DIM), gamma.reshape(1, DIM),
                      beta.reshape(1, DIM), r, w_rel)
    sub_emb, rel_emb = _lookup(x, r, subj, rel)
    return (sub_emb, rel_emb, x)


# speedup vs baseline: 7.6882x; 1.4493x over previous
import jax
import jax.numpy as jnp
from jax import lax
from jax.experimental import pallas as pl
from jax.experimental.pallas import tpu as pltpu
from jax.experimental.pallas import tpu_sc as plsc

NUM_ENT = 10000
NUM_REL2 = 400
DIM = 128
N_EDGES = 320000
HALF = N_EDGES // 2
BATCH = 1024

NC = 2
NS = 16
LN = 16

EDGES_PER_TILE = HALF // NS
CHUNK = 48
NFULL = EDGES_PER_TILE // CHUNK
REM = EDGES_PER_TILE - NFULL * CHUNK
NGRP = CHUNK // LN
ACC_ROWS = 10112
ROWS_PER_TILE = ACC_ROWS // NS

_SC_MESH = plsc.VectorSubcoreMesh(core_axis_name="c", subcore_axis_name="s")


def _mul_chunk(xrows, rrows, normb, n):
    def _group(g, _):
        nr16 = normb[pl.ds(g * LN, LN)]
        for li in range(LN):
            nrm = nr16[li]
            i = g * LN + li
            for j in range(DIM // LN):
                sl = pl.ds(j * LN, LN)
                xrows[i, sl] = xrows[i, sl] * rrows[i, sl] * nrm
        return 0

    plsc.parallel_loop(0, n // LN)(lambda g: _group(g, 0) and None)


def _edge_agg_body(src_hbm, et_hbm, norm_hbm, dst_hbm, x_hbm, r_hbm, acc_hbm,
                   src0, src1, src2, et0, et1, et2, nb0, nb1, nb2,
                   db0, db1, db2, xr0, xr1, xr2, rr0, rr1, rr2,
                   src_t, et_t, nb_t, dst_t,
                   acc_sh,
                   si0, si1, si2, sd0, sd1, sd2,
                   sx0, sx1, sx2, sr0, sr1, sr2, ss0, ss1, ss2):
    c = lax.axis_index("c")
    s = lax.axis_index("s")
    srcb = (src0, src1, src2)
    etb = (et0, et1, et2)
    nbb = (nb0, nb1, nb2)
    dbb = (db0, db1, db2)
    xrb = (xr0, xr1, xr2)
    rrb = (rr0, rr1, rr2)
    sib = (si0, si1, si2)
    sdb = (sd0, sd1, sd2)
    sxb = (sx0, sx1, sx2)
    srb = (sr0, sr1, sr2)
    ssb = (ss0, ss1, ss2)

    zero = jnp.zeros((LN,), jnp.float32)

    def _zrow(i, _):
        for j in range(DIM // LN):
            xr0[i, pl.ds(j * LN, LN)] = zero
        return 0

    lax.fori_loop(0, CHUNK, _zrow, 0)
    rbase = s * ROWS_PER_TILE
    for m in range(ROWS_PER_TILE // CHUNK):
        pltpu.sync_copy(xr0, acc_sh.at[pl.ds(rbase + m * CHUNK, CHUNK)])
    pltpu.sync_copy(xr0.at[pl.ds(0, ROWS_PER_TILE % CHUNK)],
                    acc_sh.at[pl.ds(rbase + (ROWS_PER_TILE // CHUNK) * CHUNK,
                                    ROWS_PER_TILE % CHUNK)])
    plsc.subcore_barrier()

    ebase = c * HALF + s * EDGES_PER_TILE

    def _issue_idx(k, sl):
        base = ebase + k * CHUNK
        pltpu.async_copy(src_hbm.at[pl.ds(base, CHUNK)], srcb[sl], sib[sl])
        pltpu.async_copy(et_hbm.at[pl.ds(base, CHUNK)], etb[sl], sib[sl])
        pltpu.async_copy(norm_hbm.at[pl.ds(base, CHUNK)], nbb[sl], sib[sl])

    def _wait_idx(sl):
        pltpu.make_async_copy(src_hbm.at[pl.ds(0, CHUNK)], srcb[sl], sib[sl]).wait()
        pltpu.make_async_copy(et_hbm.at[pl.ds(0, CHUNK)], etb[sl], sib[sl]).wait()
        pltpu.make_async_copy(norm_hbm.at[pl.ds(0, CHUNK)], nbb[sl], sib[sl]).wait()

    def _issue_dst(k, sl):
        pltpu.async_copy(dst_hbm.at[pl.ds(ebase + k * CHUNK, CHUNK)],
                         dbb[sl], sdb[sl])

    def _wait_dst(sl):
        pltpu.make_async_copy(dst_hbm.at[pl.ds(0, CHUNK)], dbb[sl], sdb[sl]).wait()

    def _issue_gathers(sl):
        pltpu.async_copy(x_hbm.at[srcb[sl]], xrb[sl], sxb[sl])
        pltpu.async_copy(r_hbm.at[etb[sl]], rrb[sl], srb[sl])

    def _wait_gathers(sl):
        pltpu.make_async_copy(x_hbm.at[srcb[sl]], xrb[sl], sxb[sl]).wait()
        pltpu.make_async_copy(r_hbm.at[etb[sl]], rrb[sl], srb[sl]).wait()

    def _issue_scatter(sl):
        pltpu.async_copy(xrb[sl], acc_sh.at[dbb[sl]], ssb[sl], add=True)

    def _wait_scatter(sl):
        pltpu.make_async_copy(xrb[sl], acc_sh.at[dbb[sl]], ssb[sl]).wait()

    _issue_idx(0, 0)
    _issue_idx(1, 1)
    _issue_dst(0, 0)
    _wait_idx(0)
    _issue_gathers(0)

    def _triple(it, _):
        for jj in range(3):
            s0 = jj
            s1 = (jj + 1) % 3
            s2 = (jj + 2) % 3
            kabs = 3 * it + jj

            @pl.when(kabs >= 2)
            def _():
                _wait_scatter(s1)

            @pl.when(kabs + 1 < NFULL)
            def _():
                _issue_dst(kabs + 1, s1)
                _wait_idx(s1)
                _issue_gathers(s1)

            @pl.when(kabs + 2 < NFULL)
            def _():
                _issue_idx(kabs + 2, s2)

            _wait_gathers(s0)
            _mul_chunk(xrb[s0], rrb[s0], nbb[s0], CHUNK)
            _wait_dst(s0)
            _issue_scatter(s0)
        return 0

    NCOV = (NFULL // 3) * 3
    lax.fori_loop(0, NFULL // 3, _triple, 0)

    _wait_scatter((NCOV - 2) % 3)
    _wait_scatter((NCOV - 1) % 3)

    for k in range(NCOV, NFULL):
        sl = k % 3
        _wait_gathers(sl)
        _mul_chunk(xrb[sl], rrb[sl], nbb[sl], CHUNK)
        _wait_dst(sl)
        _issue_scatter(sl)
        _wait_scatter(sl)

    tbase = ebase + NFULL * CHUNK
    pltpu.sync_copy(src_hbm.at[pl.ds(tbase, REM)], src_t)
    pltpu.sync_copy(et_hbm.at[pl.ds(tbase, REM)], et_t)
    pltpu.sync_copy(norm_hbm.at[pl.ds(tbase, REM)], nb_t)
    pltpu.sync_copy(dst_hbm.at[pl.ds(tbase, REM)], dst_t)
    pltpu.async_copy(x_hbm.at[src_t], xr0.at[pl.ds(0, REM)], sx0).wait()
    pltpu.async_copy(r_hbm.at[et_t], rr0.at[pl.ds(0, REM)], sr0).wait()
    _mul_chunk(xr0, rr0, nb_t, REM)
    pltpu.sync_copy(xr0.at[pl.ds(0, REM)], acc_sh.at[dst_t], add=True)

    plsc.subcore_barrier()
    pltpu.sync_copy(acc_sh.at[pl.ds(rbase, ROWS_PER_TILE)],
                    acc_hbm.at[c].at[pl.ds(rbase, ROWS_PER_TILE)])


_edge_agg = pl.kernel(
    _edge_agg_body,
    out_type=jax.ShapeDtypeStruct((NC, ACC_ROWS, DIM), jnp.float32),
    mesh=_SC_MESH,
    scratch_types=(
        [pltpu.VMEM((CHUNK,), jnp.int32) for _ in range(3)]
        + [pltpu.VMEM((CHUNK,), jnp.int32) for _ in range(3)]
        + [pltpu.VMEM((CHUNK,), jnp.float32) for _ in range(3)]
        + [pltpu.VMEM((CHUNK,), jnp.int32) for _ in range(3)]
        + [pltpu.VMEM((CHUNK, DIM), jnp.float32) for _ in range(3)]
        + [pltpu.VMEM((CHUNK, DIM), jnp.float32) for _ in range(3)]
        + [pltpu.VMEM((REM,), jnp.int32) for _ in range(2)]
        + [pltpu.VMEM((REM,), jnp.float32)]
        + [pltpu.VMEM((REM,), jnp.int32)]
        + [pltpu.VMEM_SHARED((ACC_ROWS, DIM), jnp.float32)]
        + [pltpu.SemaphoreType.DMA for _ in range(15)]
    ),
)


def _dense_body(acc_ref, x_ref, win_ref, wout_ref, wloop_ref, lrel_ref,
                bias_ref, gamma_ref, beta_ref, r_ref, wrel_ref,
                xout_ref, rout_ref):
    a_in = acc_ref[0]
    a_out = acc_ref[1]
    x = x_ref[...]
    pre = (jnp.dot(a_in, win_ref[...], preferred_element_type=jnp.float32)
           + jnp.dot(a_out, wout_ref[...], preferred_element_type=jnp.float32)
           + jnp.dot(x * lrel_ref[...], wloop_ref[...],
                     preferred_element_type=jnp.float32)) * (1.0 / 3.0)
    pre = pre + bias_ref[...]
    mean = jnp.mean(pre, axis=0, keepdims=True)
    var = jnp.mean((pre - mean) ** 2, axis=0, keepdims=True)
    xout_ref[...] = ((pre - mean) * lax.rsqrt(var + 1e-5) * gamma_ref[...]
                     + beta_ref[...])
    rout_ref[...] = jnp.dot(r_ref[...], wrel_ref[...],
                            preferred_element_type=jnp.float32)


_dense = pl.pallas_call(
    _dense_body,
    out_shape=(jax.ShapeDtypeStruct((NUM_ENT, DIM), jnp.float32),
               jax.ShapeDtypeStruct((NUM_REL2, DIM), jnp.float32)),
)

B_PER_W = BATCH // (NC * NS)


def _lookup_body(x_hbm, r_hbm, subj_hbm, rel_hbm, sub_out, rel_out,
                 idx_v, rows_v, sem):
    c = lax.axis_index("c")
    s = lax.axis_index("s")
    base = (s * NC + c) * B_PER_W
    pltpu.sync_copy(subj_hbm.at[pl.ds(base, B_PER_W)], idx_v)
    pltpu.async_copy(x_hbm.at[idx_v], rows_v, sem).wait()
    pltpu.sync_copy(rows_v, sub_out.at[pl.ds(base, B_PER_W)])
    pltpu.sync_copy(rel_hbm.at[pl.ds(base, B_PER_W)], idx_v)
    pltpu.async_copy(r_hbm.at[idx_v], rows_v, sem).wait()
    pltpu.sync_copy(rows_v, rel_out.at[pl.ds(base, B_PER_W)])


_lookup = pl.kernel(
    _lookup_body,
    out_type=(jax.ShapeDtypeStruct((BATCH, DIM), jnp.float32),
              jax.ShapeDtypeStruct((BATCH, DIM), jnp.float32)),
    mesh=_SC_MESH,
    scratch_types=[
        pltpu.VMEM((B_PER_W,), jnp.int32),
        pltpu.VMEM((B_PER_W, DIM), jnp.float32),
        pltpu.SemaphoreType.DMA,
    ],
)


def kernel(edge_index, subj, rel, edge_type, edge_norm, init_embed, init_rel,
           w_in_0, w_out_0, w_loop_0, w_rel_0, loop_rel_0, bias_0, gamma_0, beta_0,
           w_in_1, w_out_1, w_loop_1, w_rel_1, loop_rel_1, bias_1, gamma_1, beta_1):
    src, dst = edge_index[0], edge_index[1]
    x, r = init_embed, init_rel
    for (w_in, w_out, w_loop, w_rel, loop_rel, bias, gamma, beta) in (
            (w_in_0, w_out_0, w_loop_0, w_rel_0, loop_rel_0, bias_0, gamma_0, beta_0),
            (w_in_1, w_out_1, w_loop_1, w_rel_1, loop_rel_1, bias_1, gamma_1, beta_1)):
        acc = _edge_agg(src, edge_type, edge_norm, dst, x, r)[:, :NUM_ENT]
        x, r = _dense(acc, x, w_in, w_out, w_loop, loop_rel,
                      bias.reshape(1, DIM), gamma.reshape(1, DIM),
                      beta.reshape(1, DIM), r, w_rel)
    sub_emb, rel_emb = _lookup(x, r, subj, rel)
    return (sub_emb, rel_emb, x)
